# baseline (speedup 1.0000x reference)
import functools

import jax
import jax.numpy as jnp
from jax import lax
from jax.experimental import pallas as pl
from jax.experimental.pallas import tpu as pltpu
from jax.experimental.pallas import tpu_sc as plsc
from jax._src.pallas import mpmd as _pl_mpmd


def _distance_body(x_ref, w_ref, mean_ref, out_ref, idx_ref,
                   preds_ref, q2_ref, minv_ref, argv_ref, *, cb, c, nb):
    j = pl.program_id(0)
    b = x_ref.shape[0]

    @pl.when(j == 0)
    def _init():
        p = jnp.dot(x_ref[...], w_ref[...], preferred_element_type=jnp.float32)
        preds_ref[...] = p
        q2_ref[...] = jnp.sum(p * p, axis=1, keepdims=True)
        minv_ref[...] = jnp.full((b, 1), jnp.inf, jnp.float32)
        argv_ref[...] = jnp.zeros((b, 1), jnp.int32)

    p = preds_ref[...]
    mb = mean_ref[...]
    pm = lax.dot_general(p, mb, (((1,), (1,)), ((), ())),
                         preferred_element_type=jnp.float32)
    k2 = jnp.sum(mb * mb, axis=1)[None, :]
    d2 = (q2_ref[...] + k2) - 2.0 * pm
    d = jnp.sqrt(jnp.maximum(d2, 0.0))
    col = j * cb + lax.broadcasted_iota(jnp.int32, (b, cb), 1)
    d = jnp.where(col < c, d, jnp.inf)

    bmin = jnp.min(d, axis=1, keepdims=True)
    cand = jnp.where(d == bmin, col, jnp.int32(2**30))
    barg = jnp.min(cand, axis=1, keepdims=True)

    better = bmin < minv_ref[...]
    minv_ref[...] = jnp.where(better, bmin, minv_ref[...])
    argv_ref[...] = jnp.where(better, barg, argv_ref[...])

    out_ref[...] = jnp.zeros((b, cb), jnp.float32)

    @pl.when(j == nb - 1)
    def _fin():
        idx_ref[...] = argv_ref[...]


def _distance_argmin(x, W, mean_features, cb=1024):
    b, d = x.shape
    c = mean_features.shape[0]
    nb = (c + cb - 1) // cb
    body = functools.partial(_distance_body, cb=cb, c=c, nb=nb)
    return pl.pallas_call(
        body,
        grid=(nb,),
        in_specs=[
            pl.BlockSpec((b, d), lambda j: (0, 0)),
            pl.BlockSpec((d, d), lambda j: (0, 0)),
            pl.BlockSpec((cb, d), lambda j: (j, 0)),
        ],
        out_specs=[
            pl.BlockSpec((b, cb), lambda j: (0, j)),
            pl.BlockSpec((b, 1), lambda j: (0, 0)),
        ],
        out_shape=[
            jax.ShapeDtypeStruct((b, c), jnp.float32),
            jax.ShapeDtypeStruct((b, 1), jnp.int32),
        ],
        scratch_shapes=[
            pltpu.VMEM((b, d), jnp.float32),
            pltpu.VMEM((b, 1), jnp.float32),
            pltpu.VMEM((b, 1), jnp.float32),
            pltpu.VMEM((b, 1), jnp.int32),
        ],
    )(x, W, mean_features)


def _sc_scatter(zeros_flat, idx, b, c):
    info = plsc.get_sparse_core_info()
    nc, ns, nl = info.num_cores, info.num_subcores, info.num_lanes
    nw = nc * ns
    rw = b // nw

    def body(zeros_hbm, idx_hbm, out_hbm, idxv, flatv, onesv, sem):
        del zeros_hbm
        wid = lax.axis_index("s") * nc + lax.axis_index("c")
        base = wid * rw
        pltpu.sync_copy(idx_hbm.at[pl.ds(base, rw)], idxv)
        for i in range(rw // nl):
            v = idxv[pl.ds(i * nl, nl)]
            row = lax.iota(jnp.int32, nl) + (base + i * nl)
            flatv[pl.ds(i * nl, nl)] = row * c + v
            onesv[pl.ds(i * nl, nl)] = jnp.full((nl,), 1.0, jnp.float32)
        pltpu.async_copy(onesv, out_hbm.at[flatv], sem).wait()

    mesh = plsc.VectorSubcoreMesh(core_axis_name="c", subcore_axis_name="s")
    run = _pl_mpmd._mpmd_map(
        [(mesh, body)],
        [jax.ShapeDtypeStruct((b * c,), jnp.float32)],
        input_output_aliases={0: 0},
        scratch_types=[
            pltpu.VMEM((rw,), jnp.int32),
            pltpu.VMEM((rw,), jnp.int32),
            pltpu.VMEM((rw,), jnp.float32),
            pltpu.SemaphoreType.DMA,
        ],
        name="one_hot_scatter",
    )
    return run(zeros_flat, idx)[0]


def kernel(x, W, mean_features):
    b = x.shape[0]
    c = mean_features.shape[0]
    zeros, idx = _distance_argmin(x, W, mean_features)
    out_flat = _sc_scatter(zeros.reshape(b * c), idx.reshape(b), b, c)
    return out_flat.reshape(b, c)


# speedup vs baseline: 4.1736x; 4.1736x over previous
import functools

import jax
import jax.numpy as jnp
from jax import lax
from jax.experimental import pallas as pl
from jax.experimental.pallas import tpu as pltpu


def _distance_body(x_ref, w_ref, meant_ref, idx_ref,
                   preds2_ref, q2_ref, minv_ref, argv_ref, *, cb, c, nb):
    j = pl.program_id(0)
    b = x_ref.shape[0]

    @pl.when(j == 0)
    def _init():
        p = jnp.dot(x_ref[...], w_ref[...], preferred_element_type=jnp.float32)
        preds2_ref[...] = p + p
        q2_ref[...] = jnp.sum(p * p, axis=1, keepdims=True)
        minv_ref[...] = jnp.full((b, 1), jnp.inf, jnp.float32)
        argv_ref[...] = jnp.zeros((b, 1), jnp.int32)

    mt = meant_ref[...]
    pm2 = jnp.dot(preds2_ref[...], mt, preferred_element_type=jnp.float32)
    k2 = jnp.sum(mt * mt, axis=0, keepdims=True)
    col = j * cb + lax.broadcasted_iota(jnp.int32, (1, cb), 1)

    def block_minarg(d):
        bmin = jnp.min(d, axis=1, keepdims=True)
        cand = jnp.where(d == bmin, jnp.broadcast_to(col, d.shape),
                         jnp.int32(2**30))
        return bmin, jnp.min(cand, axis=1, keepdims=True)

    def merge(bmin, barg):
        better = bmin < minv_ref[...]
        minv_ref[...] = jnp.where(better, bmin, minv_ref[...])
        argv_ref[...] = jnp.where(better, barg, argv_ref[...])

    @pl.when(j < nb - 1)
    def _full_block():
        d2 = (q2_ref[...] + k2) - pm2
        d = jnp.sqrt(jnp.maximum(d2, 0.0))
        merge(*block_minarg(d))

    @pl.when(j == nb - 1)
    def _last_block():
        d2 = (q2_ref[...] + k2) - pm2
        d = jnp.sqrt(jnp.maximum(d2, 0.0))
        d = jnp.where(jnp.broadcast_to(col, d.shape) < c, d, jnp.inf)
        merge(*block_minarg(d))
        idx_ref[...] = argv_ref[...]


def _distance_argmin(x, W, mean_t, cb):
    b, d = x.shape
    c = mean_t.shape[1]
    nb = (c + cb - 1) // cb
    body = functools.partial(_distance_body, cb=cb, c=c, nb=nb)
    return pl.pallas_call(
        body,
        grid=(nb,),
        in_specs=[
            pl.BlockSpec((b, d), lambda j: (0, 0)),
            pl.BlockSpec((d, d), lambda j: (0, 0)),
            pl.BlockSpec((d, cb), lambda j: (0, j)),
        ],
        out_specs=pl.BlockSpec((b, 1), lambda j: (0, 0)),
        out_shape=jax.ShapeDtypeStruct((b, 1), jnp.int32),
        scratch_shapes=[
            pltpu.VMEM((b, d), jnp.float32),
            pltpu.VMEM((b, 1), jnp.float32),
            pltpu.VMEM((b, 1), jnp.float32),
            pltpu.VMEM((b, 1), jnp.int32),
        ],
    )(x, W, mean_t)


def _onehot_body(idx_ref, out_ref, *, cb):
    j = pl.program_id(0)
    rows = j * cb + lax.broadcasted_iota(jnp.int32, (cb, 1), 0)
    onehot = rows == idx_ref[...]
    out_ref[...] = onehot.astype(jnp.float32)


def _onehot_t(idx_row, c, cb):
    b = idx_row.shape[1]
    nb = (c + cb - 1) // cb
    body = functools.partial(_onehot_body, cb=cb)
    return pl.pallas_call(
        body,
        grid=(nb,),
        in_specs=[pl.BlockSpec((1, b), lambda j: (0, 0))],
        out_specs=pl.BlockSpec((cb, b), lambda j: (j, 0)),
        out_shape=jax.ShapeDtypeStruct((c, b), jnp.float32),
    )(idx_row)


def kernel(x, W, mean_features):
    b = x.shape[0]
    c = mean_features.shape[0]
    idx = _distance_argmin(x, W, mean_features.T, cb=1024)
    out_t = _onehot_t(idx.reshape(1, b), c, cb=2048)
    return out_t.T


# speedup vs baseline: 5.9452x; 1.4245x over previous
import functools

import jax
import jax.numpy as jnp
from jax import lax
from jax.experimental import pallas as pl
from jax.experimental.pallas import tpu as pltpu


_LANES = 128


def _distance_body(x_ref, w_ref, meant_ref, idx_ref,
                   preds2_ref, q2_ref, accv_ref, acci_ref, *, cb, c, nb):
    j = pl.program_id(0)
    b = x_ref.shape[0]
    ng = cb // _LANES

    @pl.when(j == 0)
    def _init():
        p = jnp.dot(x_ref[...], w_ref[...], preferred_element_type=jnp.float32)
        preds2_ref[...] = p + p
        q2_ref[...] = jnp.sum(p * p, axis=1, keepdims=True)
        accv_ref[...] = jnp.full((b, _LANES), jnp.inf, jnp.float32)
        acci_ref[...] = jnp.zeros((b, _LANES), jnp.int32)

    mt = meant_ref[...]
    pm2 = jnp.dot(preds2_ref[...], mt, preferred_element_type=jnp.float32)
    k2 = jnp.sum(mt * mt, axis=0, keepdims=True)
    q2b = jnp.broadcast_to(q2_ref[...], (b, _LANES))

    def scan_block(masked):
        m = None
        gv = None
        for g in range(ng):
            sl = slice(g * _LANES, (g + 1) * _LANES)
            d2g = (q2b + k2[:, sl]) - pm2[:, sl]
            if masked:
                colg = (j * cb + g * _LANES
                        + lax.broadcasted_iota(jnp.int32, (1, _LANES), 1))
                d2g = jnp.where(jnp.broadcast_to(colg < c, d2g.shape),
                                d2g, jnp.inf)
            if g == 0:
                m = d2g
                gv = jnp.zeros((b, _LANES), jnp.int32)
            else:
                better = d2g < m
                m = jnp.where(better, d2g, m)
                gv = jnp.where(better, jnp.int32(g), gv)
        better = m < accv_ref[...]
        accv_ref[...] = jnp.where(better, m, accv_ref[...])
        acci_ref[...] = jnp.where(better, j * ng + gv, acci_ref[...])

    @pl.when(j < nb - 1)
    def _full_block():
        scan_block(False)

    @pl.when(j == nb - 1)
    def _last_block():
        scan_block(True)
        sv = jnp.sqrt(jnp.maximum(accv_ref[...], 0.0))
        cols = acci_ref[...] * _LANES + lax.broadcasted_iota(
            jnp.int32, (b, _LANES), 1)
        svmin = jnp.min(sv, axis=1, keepdims=True)
        cand = jnp.where(sv == svmin, cols, jnp.int32(2**30))
        idx_ref[...] = jnp.min(cand, axis=1, keepdims=True)


def _distance_argmin(x, W, mean_t, cb):
    b, d = x.shape
    c = mean_t.shape[1]
    nb = (c + cb - 1) // cb
    body = functools.partial(_distance_body, cb=cb, c=c, nb=nb)
    return pl.pallas_call(
        body,
        grid=(nb,),
        in_specs=[
            pl.BlockSpec((b, d), lambda j: (0, 0)),
            pl.BlockSpec((d, d), lambda j: (0, 0)),
            pl.BlockSpec((d, cb), lambda j: (0, j)),
        ],
        out_specs=pl.BlockSpec((b, 1), lambda j: (0, 0)),
        out_shape=jax.ShapeDtypeStruct((b, 1), jnp.int32),
        scratch_shapes=[
            pltpu.VMEM((b, d), jnp.float32),
            pltpu.VMEM((b, 1), jnp.float32),
            pltpu.VMEM((b, _LANES), jnp.float32),
            pltpu.VMEM((b, _LANES), jnp.int32),
        ],
    )(x, W, mean_t)


def _onehot_body(idx_ref, out_ref, *, cb):
    j = pl.program_id(0)
    rows = j * cb + lax.broadcasted_iota(jnp.int32, (cb, 1), 0)
    onehot = rows == idx_ref[...]
    out_ref[...] = onehot.astype(jnp.float32)


def _onehot_t(idx_row, c, cb):
    b = idx_row.shape[1]
    nb = (c + cb - 1) // cb
    body = functools.partial(_onehot_body, cb=cb)
    return pl.pallas_call(
        body,
        grid=(nb,),
        in_specs=[pl.BlockSpec((1, b), lambda j: (0, 0))],
        out_specs=pl.BlockSpec((cb, b), lambda j: (j, 0)),
        out_shape=jax.ShapeDtypeStruct((c, b), jnp.float32),
    )(idx_row)


def kernel(x, W, mean_features):
    b = x.shape[0]
    c = mean_features.shape[0]
    idx = _distance_argmin(x, W, mean_features.T, cb=1024)
    out_t = _onehot_t(idx.reshape(1, b), c, cb=2048)
    return out_t.T
